# baseline (speedup 1.0000x reference)
import functools

import jax
import jax.numpy as jnp
from jax import lax
from jax.experimental import pallas as pl
from jax.experimental.pallas import tpu as pltpu
from jax.experimental.pallas import tpu_sc as plsc

_DIM = 64
_PDIM = 128
_IDX_LANES = 128
_HALF = 256


@functools.cache
def _build_gather(batch, fields):
    info = plsc.get_sparse_core_info()
    nc, ns = info.num_cores, info.num_subcores
    nw = nc * ns
    bpw = batch // nw
    halves = bpw // _HALF
    steps = fields * halves
    tiles_b = batch // 128
    tiles_half = _HALF // 128

    mesh = plsc.VectorSubcoreMesh(core_axis_name="c", subcore_axis_name="s")

    @functools.partial(
        pl.kernel,
        mesh=mesh,
        compiler_params=pltpu.CompilerParams(
            use_tc_tiling_on_sc=False, needs_layout_passes=False
        ),
        out_type=jax.ShapeDtypeStruct(
            (fields, _DIM // 8, tiles_b, 8, 128), jnp.float32
        ),
        scratch_types=[
            pltpu.VMEM((fields, bpw), jnp.int32),
            pltpu.VMEM((2, _HALF, _PDIM), jnp.float32),
            pltpu.VMEM((2, _DIM // 8, tiles_half, 8, 128), jnp.float32),
            pltpu.SemaphoreType.DMA((2,)),
            pltpu.SemaphoreType.DMA((2,)),
        ],
    )
    def gather_kernel(xt_hbm, table_hbm, out_hbm, idx_v, rv, tv, gsem, ssem):
        wid = lax.axis_index("s") * nc + lax.axis_index("c")
        b0 = wid * bpw
        bg0 = wid * (bpw // 128)
        pltpu.sync_copy(xt_hbm.at[:, pl.ds(b0, bpw)], idx_v)

        iota = lax.iota(jnp.int32, 16)

        def start_gather(f, half, p):
            for j in range(_HALF // _IDX_LANES):
                pltpu.async_copy(
                    table_hbm.at[
                        idx_v.at[f, pl.ds(half * _HALF + j * _IDX_LANES,
                                          _IDX_LANES)]
                    ],
                    rv.at[p, pl.ds(j * _IDX_LANES, _IDX_LANES)],
                    gsem.at[p],
                )

        def wait_gather(p):
            pltpu.make_async_copy(
                table_hbm.at[pl.ds(0, _HALF)], rv.at[p], gsem.at[p]
            ).wait()

        def store_dst(f, half):
            return out_hbm.at[f, :, pl.ds(bg0 + half * tiles_half, tiles_half)]

        def wait_store(p):
            pltpu.make_async_copy(tv.at[p], store_dst(0, 0), ssem.at[p]).wait()

        start_gather(0, 0, 0)

        @pl.loop(0, steps, step=2)
        def _steps(lv):
            f = lv // halves
            for p in range(2):
                s = lv + p

                @pl.when(s < steps - 1)
                def _():
                    start_gather((s + 1) // halves, (s + 1) % halves, 1 - p)

                wait_gather(p)

                @pl.when(s >= 2)
                def _():
                    wait_store(p)

                @pl.loop(0, _DIM)
                def _cols(c):
                    cg = c // 8
                    ci = c % 8
                    csplat = jnp.full((16,), c, jnp.int32)
                    for l in range(tiles_half):
                        for k in range(128 // 16):
                            vals = plsc.load_gather(
                                rv.at[p],
                                [iota + (l * 128 + k * 16), csplat],
                            )
                            tv[p, cg, l, ci, pl.ds(k * 16, 16)] = vals

                pltpu.async_copy(tv.at[p], store_dst(f, p), ssem.at[p])

        for p in range(2):
            wait_store(p)

    return gather_kernel


def kernel(x, weight):
    b, f = x.shape
    xt = jnp.swapaxes(x, 0, 1).astype(jnp.int32)
    w128 = jnp.pad(weight, ((0, 0), (0, _PDIM - _DIM)))
    out5 = _build_gather(b, f)(xt, w128)
    return jnp.transpose(out5, (2, 4, 0, 1, 3)).reshape(b, f, _DIM)
